# baseline (speedup 1.0000x reference)
import functools

import jax
import jax.numpy as jnp
from jax import lax
from jax.experimental import pallas as pl
from jax.experimental.pallas import tpu as pltpu
from jax.experimental.pallas import tpu_sc as plsc

NUM_USERS = 1000000
NUM_MOVIES = 100000
EMBED_DIM = 64
BATCH = 16384

NC = 2
NS = 16
NW = NC * NS
B_PER_W = BATCH // NW
N_CHUNK = 4
CHUNK = B_PER_W // N_CHUNK


def _sc_gather(uid2d, mid2d, user_table, movie_table):
    mesh = plsc.VectorSubcoreMesh(core_axis_name="c", subcore_axis_name="s")

    @functools.partial(
        pl.kernel,
        out_type=(
            jax.ShapeDtypeStruct((NW * N_CHUNK, CHUNK, EMBED_DIM), jnp.float32),
            jax.ShapeDtypeStruct((NW * N_CHUNK, CHUNK, EMBED_DIM), jnp.float32),
        ),
        mesh=mesh,
        scratch_types=[
            pltpu.VMEM((N_CHUNK, CHUNK), jnp.int32),
            pltpu.VMEM((N_CHUNK, CHUNK), jnp.int32),
            pltpu.VMEM((N_CHUNK, CHUNK, EMBED_DIM), jnp.float32),
            pltpu.VMEM((N_CHUNK, CHUNK, EMBED_DIM), jnp.float32),
            pltpu.SemaphoreType.DMA,
        ],
        compiler_params=pltpu.CompilerParams(use_tc_tiling_on_sc=False),
    )
    def k(uid_hbm, mid_hbm, ut_hbm, mt_hbm, u_out, m_out, idxu_v, idxm_v,
          rowsu_v, rowsm_v, sem):
        wid = lax.axis_index("s") * NC + lax.axis_index("c")
        base = wid * N_CHUNK
        pltpu.sync_copy(uid_hbm.at[pl.ds(base, N_CHUNK)], idxu_v)
        pltpu.sync_copy(mid_hbm.at[pl.ds(base, N_CHUNK)], idxm_v)
        copies = []
        for j in range(N_CHUNK):
            copies.append(
                pltpu.async_copy(ut_hbm.at[idxu_v.at[j]], rowsu_v.at[j], sem))
            copies.append(
                pltpu.async_copy(mt_hbm.at[idxm_v.at[j]], rowsm_v.at[j], sem))
        for c in copies:
            c.wait()
        pltpu.sync_copy(rowsu_v, u_out.at[pl.ds(base, N_CHUNK)])
        pltpu.sync_copy(rowsm_v, m_out.at[pl.ds(base, N_CHUNK)])

    return k(uid2d, mid2d, user_table, movie_table)


BR = 2048


def _mlp_body(u_ref, m_ref, w1_ref, b1_ref, w2_ref, b2_ref, o_ref):
    x = jnp.dot(u_ref[...], w1_ref[0:EMBED_DIM, :],
                preferred_element_type=jnp.float32)
    x += jnp.dot(m_ref[...], w1_ref[EMBED_DIM:, :],
                 preferred_element_type=jnp.float32)
    h = jnp.maximum(x + b1_ref[...], 0.0)
    o = jnp.dot(h, w2_ref[...], preferred_element_type=jnp.float32) + b2_ref[...]
    o_ref[...] = jax.nn.sigmoid(o) * 4.0 + 1.0


def _tc_mlp(u, m, W1, b1, W2, b2):
    grid = (BATCH // BR,)
    return pl.pallas_call(
        _mlp_body,
        grid=grid,
        in_specs=[
            pl.BlockSpec((BR, EMBED_DIM), lambda i: (i, 0)),
            pl.BlockSpec((BR, EMBED_DIM), lambda i: (i, 0)),
            pl.BlockSpec((2 * EMBED_DIM, 8), lambda i: (0, 0)),
            pl.BlockSpec((1, 8), lambda i: (0, 0)),
            pl.BlockSpec((8, 1), lambda i: (0, 0)),
            pl.BlockSpec((1, 1), lambda i: (0, 0)),
        ],
        out_specs=pl.BlockSpec((BR, 1), lambda i: (i, 0)),
        out_shape=jax.ShapeDtypeStruct((BATCH, 1), jnp.float32),
    )(u, m, W1, b1, W2, b2)


def kernel(user_ids, movie_ids, user_table, movie_table, W1, b1, W2, b2):
    uid2d = user_ids.reshape(NW * N_CHUNK, CHUNK).astype(jnp.int32)
    mid2d = movie_ids.reshape(NW * N_CHUNK, CHUNK).astype(jnp.int32)
    u3, m3 = _sc_gather(uid2d, mid2d, user_table, movie_table)
    u = u3.reshape(BATCH, EMBED_DIM)
    m = m3.reshape(BATCH, EMBED_DIM)
    out = _tc_mlp(u, m, W1, b1.reshape(1, 8), W2, b2.reshape(1, 1))
    return out.reshape(BATCH)
